# baseline (speedup 1.0000x reference)
import functools

import jax
import jax.numpy as jnp
from jax.experimental import pallas as pl
from jax.experimental.pallas import tpu as pltpu

N = 4096
D = 32
K = 8192
GAMMA = 0.1
EPS = 1e-6

TN = 512
KC = 2048


def _dce_body(feat_ref, label_ref, protos_t_ref, plabels_ref, out_ref):
    xe = feat_ref[...] + EPS
    x2 = jnp.sum(xe * xe, axis=1, keepdims=True)
    lab = label_ref[...]

    denom = jnp.zeros((TN, 1), jnp.float32)
    numer = jnp.zeros((TN, 1), jnp.float32)
    for c in range(K // KC):
        pt = protos_t_ref[:, c * KC:(c + 1) * KC]
        p2 = jnp.sum(pt * pt, axis=0, keepdims=True)
        d2 = x2 - 2.0 * jnp.dot(xe, pt, preferred_element_type=jnp.float32) + p2
        d = jnp.sqrt(jnp.maximum(d2, 0.0))
        e = jnp.exp(-GAMMA * d)
        denom = denom + jnp.sum(e, axis=1, keepdims=True)
        mask = plabels_ref[:, c * KC:(c + 1) * KC] == lab
        numer = numer + jnp.sum(jnp.where(mask, e, 0.0), axis=1, keepdims=True)

    prob = jnp.where(denom > 0.0, numer / denom, numer + 1e-6)
    out_ref[...] = -jnp.log(prob)


@jax.jit
def kernel(feature, label, prototypes, proto_labels):
    protos_t = prototypes.T
    label2 = label.astype(jnp.int32).reshape(N, 1)
    plabels2 = proto_labels.astype(jnp.int32).reshape(1, K)

    out = pl.pallas_call(
        _dce_body,
        grid=(N // TN,),
        in_specs=[
            pl.BlockSpec((TN, D), lambda i: (i, 0)),
            pl.BlockSpec((TN, 1), lambda i: (i, 0)),
            pl.BlockSpec((D, K), lambda i: (0, 0)),
            pl.BlockSpec((1, K), lambda i: (0, 0)),
        ],
        out_specs=pl.BlockSpec((TN, 1), lambda i: (i, 0)),
        out_shape=jax.ShapeDtypeStruct((N, 1), jnp.float32),
    )(feature, label2, protos_t, plabels2)
    return out.reshape(N)


# speedup vs baseline: 1.3013x; 1.3013x over previous
import jax
import jax.numpy as jnp
from jax.experimental import pallas as pl
from jax.experimental.pallas import tpu as pltpu

N = 4096
D = 32
K = 8192
NUM_CLASSES = 100
GAMMA = 0.1
EPS = 1e-6
LOG2E = 1.4426950408889634
C = (GAMMA * LOG2E) ** 2

TN = 512
KC = 2048
CPAD = 128


def _dce_body(feat_ref, label_ref, protos_t_ref, plabels_ref, out_ref,
              pa_ref, w_ref):
    @pl.when(pl.program_id(0) == 0)
    def _build_tables():
        pt = protos_t_ref[...]
        p2 = jnp.sum(pt * pt, axis=0, keepdims=True) * C
        ones = jnp.ones((1, K), jnp.float32)
        pa_ref[...] = jnp.concatenate([pt, ones, p2], axis=0)
        cls = jax.lax.broadcasted_iota(jnp.int32, (1, CPAD), 1)
        w_ref[...] = (plabels_ref[...] == cls).astype(jnp.bfloat16)

    xe = feat_ref[...] + EPS
    x2 = jnp.sum(xe * xe, axis=1, keepdims=True) * C
    xa = jnp.concatenate(
        [xe * (-2.0 * C), x2, jnp.ones((TN, 1), jnp.float32)], axis=1)

    S = jnp.zeros((TN, CPAD), jnp.float32)
    for c in range(K // KC):
        t = jnp.dot(xa, pa_ref[:, c * KC:(c + 1) * KC],
                    preferred_element_type=jnp.float32)
        s = jnp.sqrt(jnp.maximum(t, 0.0))
        e = jnp.exp2(-s).astype(jnp.bfloat16)
        S = S + jnp.dot(e, w_ref[c * KC:(c + 1) * KC, :],
                        preferred_element_type=jnp.float32)

    lab = label_ref[...]
    cls = jax.lax.broadcasted_iota(jnp.int32, (1, CPAD), 1)
    numer = jnp.sum(jnp.where(lab == cls, S, 0.0), axis=1, keepdims=True)
    denom = jnp.sum(S, axis=1, keepdims=True)
    prob = jnp.where(denom > 0.0, numer / denom, numer + 1e-6)
    out_ref[...] = -jnp.log(prob)


@jax.jit
def kernel(feature, label, prototypes, proto_labels):
    protos_t = prototypes.T
    label2 = label.astype(jnp.int32).reshape(N, 1)
    plabels2 = proto_labels.astype(jnp.int32).reshape(K, 1)

    out = pl.pallas_call(
        _dce_body,
        grid=(N // TN,),
        in_specs=[
            pl.BlockSpec((TN, D), lambda i: (i, 0)),
            pl.BlockSpec((TN, 1), lambda i: (i, 0)),
            pl.BlockSpec((D, K), lambda i: (0, 0)),
            pl.BlockSpec((K, 1), lambda i: (0, 0)),
        ],
        out_specs=pl.BlockSpec((TN, 1), lambda i: (i, 0)),
        out_shape=jax.ShapeDtypeStruct((N, 1), jnp.float32),
        scratch_shapes=[
            pltpu.VMEM((D + 2, K), jnp.float32),
            pltpu.VMEM((K, CPAD), jnp.bfloat16),
        ],
    )(feature, label2, protos_t, plabels2)
    return out.reshape(N)


# speedup vs baseline: 1.5347x; 1.1793x over previous
import jax
import jax.numpy as jnp
from jax.experimental import pallas as pl
from jax.experimental.pallas import tpu as pltpu

N = 4096
D = 32
K = 8192
NUM_CLASSES = 100
GAMMA = 0.1
EPS = 1e-6
LOG2E = 1.4426950408889634
C = (GAMMA * LOG2E) ** 2

TN = 512
KC = 2048
CPAD = 128


def _dce_body(feat_ref, label_ref, protos_t_ref, plabels_ref, out_ref,
              pa_ref, w_ref):
    @pl.when(pl.program_id(0) == 0)
    def _build_tables():
        pt = protos_t_ref[...]
        p2 = jnp.sum(pt * pt, axis=0, keepdims=True) * C
        ones = jnp.ones((1, K), jnp.float32)
        pa_ref[...] = jnp.concatenate([pt, ones, p2], axis=0)
        cls = jax.lax.broadcasted_iota(jnp.int32, (1, CPAD), 1)
        w_ref[...] = (plabels_ref[...] == cls).astype(jnp.bfloat16)

    xe = feat_ref[...] + EPS
    x2 = jnp.dot(xe * xe, jnp.full((D, 1), C, jnp.float32),
                 preferred_element_type=jnp.float32)
    xa = jnp.concatenate(
        [xe * (-2.0 * C), x2, jnp.ones((TN, 1), jnp.float32)], axis=1)

    S = jnp.zeros((TN, CPAD), jnp.float32)
    for c in range(K // KC):
        t = jnp.dot(xa, pa_ref[:, c * KC:(c + 1) * KC],
                    preferred_element_type=jnp.float32)
        u = jnp.maximum(t, 1e-30)
        s = u * jax.lax.rsqrt(u)
        e = jnp.exp2(0.0 - s).astype(jnp.bfloat16)
        S = S + jnp.dot(e, w_ref[c * KC:(c + 1) * KC, :],
                        preferred_element_type=jnp.float32)

    lab = label_ref[...]
    cls = jax.lax.broadcasted_iota(jnp.int32, (1, CPAD), 1)
    numer = jnp.sum(jnp.where(lab == cls, S, 0.0), axis=1, keepdims=True)
    denom = jnp.sum(S, axis=1, keepdims=True)
    prob = jnp.where(denom > 0.0, numer / denom, numer + 1e-6)
    out_ref[...] = -jnp.log(prob)


@jax.jit
def kernel(feature, label, prototypes, proto_labels):
    protos_t = prototypes.T
    label2 = label.astype(jnp.int32).reshape(N, 1)
    plabels2 = proto_labels.astype(jnp.int32).reshape(K, 1)

    out = pl.pallas_call(
        _dce_body,
        grid=(N // TN,),
        in_specs=[
            pl.BlockSpec((TN, D), lambda i: (i, 0)),
            pl.BlockSpec((TN, 1), lambda i: (i, 0)),
            pl.BlockSpec((D, K), lambda i: (0, 0)),
            pl.BlockSpec((K, 1), lambda i: (0, 0)),
        ],
        out_specs=pl.BlockSpec((TN, 1), lambda i: (i, 0)),
        out_shape=jax.ShapeDtypeStruct((N, 1), jnp.float32),
        scratch_shapes=[
            pltpu.VMEM((D + 2, K), jnp.float32),
            pltpu.VMEM((K, CPAD), jnp.bfloat16),
        ],
    )(feature, label2, protos_t, plabels2)
    return out.reshape(N)
